# baseline (speedup 1.0000x reference)
import functools

import jax
import jax.numpy as jnp
from jax import lax
from jax.experimental import pallas as pl
from jax.experimental.pallas import tpu as pltpu
from jax.experimental.pallas import tpu_sc as plsc

Q = 64
D = 32
BK = 8000


def _dist_argmax_body(x2_ref, p_ref, bi_ref, bs_ref):
    i = pl.program_id(0)

    @pl.when(i == 0)
    def _init():
        bs_ref[...] = jnp.full((Q,), -jnp.inf, jnp.float32)
        bi_ref[...] = jnp.zeros((Q,), jnp.int32)

    p = p_ref[...]
    cross2 = lax.dot_general(
        x2_ref[...], p, (((1,), (1,)), ((), ())),
        preferred_element_type=jnp.float32,
    )
    p2 = jnp.sum(p * p, axis=1)
    s = cross2 - p2[None, :]
    m = jnp.max(s, axis=1)
    li = jnp.argmax(s, axis=1).astype(jnp.int32)
    gi = i * BK + li
    upd = m > bs_ref[...]
    bs_ref[...] = jnp.where(upd, m, bs_ref[...])
    bi_ref[...] = jnp.where(upd, gi, bi_ref[...])


def _dist_argmax(x2, prototypes):
    k = prototypes.shape[0]
    grid = k // BK
    return pl.pallas_call(
        _dist_argmax_body,
        grid=(grid,),
        in_specs=[
            pl.BlockSpec((Q, D), lambda i: (0, 0)),
            pl.BlockSpec((BK, D), lambda i: (i, 0)),
        ],
        out_specs=[
            pl.BlockSpec((Q,), lambda i: (0,)),
            pl.BlockSpec((Q,), lambda i: (0,)),
        ],
        out_shape=[
            jax.ShapeDtypeStruct((Q,), jnp.int32),
            jax.ShapeDtypeStruct((Q,), jnp.float32),
        ],
        compiler_params=pltpu.CompilerParams(
            dimension_semantics=("arbitrary",),
        ),
    )(x2, prototypes)


@functools.cache
def _make_sc_gather():
    mesh = plsc.VectorSubcoreMesh(core_axis_name="c", subcore_axis_name="s")

    @functools.partial(
        pl.kernel,
        mesh=mesh,
        out_type=jax.ShapeDtypeStruct((Q,), jnp.float32),
        scratch_types=[
            pltpu.VMEM((Q,), jnp.int32),
            pltpu.VMEM((Q,), jnp.int32),
            pltpu.VMEM((Q,), jnp.float32),
            pltpu.SemaphoreType.DMA,
        ],
    )
    def sc_gather(labels_hbm, idx_hbm, out_hbm, idx_v, lab_v, labf_v, sem):
        cid = lax.axis_index("c")
        sid = lax.axis_index("s")

        @pl.when(jnp.logical_and(cid == 0, sid == 0))
        def _():
            pltpu.sync_copy(idx_hbm, idx_v)
            pltpu.async_copy(labels_hbm.at[idx_v], lab_v, sem).wait()
            for j in range(Q // 16):
                sl = pl.ds(j * 16, 16)
                labf_v[sl] = lab_v[sl].astype(jnp.float32)
            pltpu.sync_copy(labf_v, out_hbm)

    return sc_gather


@jax.jit
def kernel(x, preds, prototypes, labels):
    x2 = x + x
    best_i, _ = _dist_argmax(x2, prototypes)
    lab = _make_sc_gather()(labels, best_i)
    return preds.at[:, -1].set(lab)
